# baseline (speedup 1.0000x reference)
import functools

import jax
import jax.numpy as jnp
from jax import lax
from jax.experimental import pallas as pl
from jax.experimental.pallas import tpu as pltpu
from jax.experimental.pallas import tpu_sc as plsc

LANES = 16
NC, NS = 2, 16
NW = NC * NS
QROWS = 64
NSTREAM = 4
WAVE = QROWS * NSTREAM
HALF = WAVE // 2


def _rsqrt(v):
    bits = lax.bitcast_convert_type(v, jnp.int32)
    y = lax.bitcast_convert_type(jnp.int32(0x5F3759DF) - (bits >> 1), jnp.float32)
    for _ in range(3):
        y = y * (jnp.float32(1.5) - jnp.float32(0.5) * v * y * y)
    return y


def _make_sc_kernel(n_tokens, vocab, emb, n_pos):
    per_w = n_tokens // NW
    n_waves = per_w // WAVE
    groups_per_half = HALF // LANES
    nsub = emb // LANES
    assert per_w % WAVE == 0 and n_waves % 2 == 0
    mesh = plsc.VectorSubcoreMesh(core_axis_name="c", subcore_axis_name="s")

    @functools.partial(
        pl.kernel,
        out_type=jax.ShapeDtypeStruct((n_tokens * emb,), jnp.float32),
        mesh=mesh,
        compiler_params=pltpu.CompilerParams(
            use_tc_tiling_on_sc=False, needs_layout_passes=False),
        scratch_types=[
            pltpu.VMEM((WAVE,), jnp.int32),
            pltpu.VMEM((WAVE,), jnp.int32),
            pltpu.VMEM((WAVE,), jnp.int32),
            pltpu.VMEM((WAVE,), jnp.int32),
            pltpu.VMEM((WAVE,), jnp.int32),
            pltpu.VMEM((WAVE, emb), jnp.float32),
            pltpu.VMEM((WAVE, emb), jnp.float32),
            pltpu.VMEM((HALF * emb,), jnp.float32),
            pltpu.VMEM((HALF * emb,), jnp.float32),
            pltpu.VMEM((n_pos, emb), jnp.float32),
            pltpu.VMEM((emb,), jnp.float32),
            pltpu.VMEM((emb,), jnp.float32),
            [[pltpu.SemaphoreType.DMA] * NSTREAM] * 2,
            [pltpu.SemaphoreType.DMA] * 2,
        ],
    )
    def sc_kernel(src_h, seg_h, word_h, pos_h, g_h, b_h, out_h,
                  idx0, idx1, segb, pidx0, pidx1, wb0, wb1,
                  obuf0, obuf1, posv, gv, bv, gsems, ssems):
        wid = lax.axis_index("s") * NC + lax.axis_index("c")
        pltpu.sync_copy(pos_h, posv)
        pltpu.sync_copy(g_h, gv)
        pltpu.sync_copy(b_h, bv)
        base0 = wid * per_w
        gs = [gv[pl.ds(LANES * k, LANES)] for k in range(nsub)]
        bs = [bv[pl.ds(LANES * k, LANES)] for k in range(nsub)]
        inv_e = jnp.float32(1.0 / emb)
        iota = lax.iota(jnp.int32, LANES)
        idxs = (idx0, idx1)
        pidxs = (pidx0, pidx1)
        banks = (wb0, wb1)
        obufs = (obuf0, obuf1)

        def gather_q(p, q):
            return pltpu.make_async_copy(
                word_h.at[idxs[p].at[pl.ds(q * QROWS, QROWS)]],
                banks[p].at[pl.ds(q * QROWS, QROWS)],
                gsems[p][q])

        def store_h(h, w):
            base = (base0 + w * WAVE + h * HALF) * emb
            return pltpu.make_async_copy(
                obufs[h], out_h.at[pl.ds(base, HALF * emb)], ssems[h])

        def stage(w, p):
            base = base0 + w * WAVE
            pltpu.sync_copy(src_h.at[pl.ds(base, WAVE)], idxs[p])
            pltpu.sync_copy(seg_h.at[pl.ds(base, WAVE)], segb)
            pidx_r = pidxs[p]

            def pix_body(j):
                s = segb[pl.ds(j * LANES, LANES)]
                q = (s.astype(jnp.float32) * jnp.float32(1.0 / 10000.0)).astype(jnp.int32)
                r = s - q * 10000
                q = jnp.where(r >= 10000, q + 1, q)
                q = jnp.where(r < 0, q - 1, q)
                pidx_r[pl.ds(j * LANES, LANES)] = q

            plsc.parallel_loop(0, WAVE // LANES, 1)(pix_body)

        def compute_half(p, h):
            wbuf = banks[p]
            pidx_r = pidxs[p]
            obuf = obufs[h]

            def group_body(g):
                lrow = g * LANES
                wrow = h * HALF + lrow
                rowv = iota + wrow
                oflat = (iota + lrow) * emb
                pvec = pidx_r[pl.ds(wrow, LANES)]
                acc = [jnp.zeros((LANES,), jnp.float32) for _ in range(4)]
                acc2 = [jnp.zeros((LANES,), jnp.float32) for _ in range(4)]
                for d in range(emb):
                    dskew = (iota + d) & (emb - 1)
                    x = (plsc.load_gather(wbuf, [rowv, dskew])
                         + plsc.load_gather(posv, [pvec, dskew]))
                    acc[d % 4] = acc[d % 4] + x
                    acc2[d % 4] = acc2[d % 4] + x * x
                    plsc.store_scatter(obuf, [oflat + dskew], x)
                sumv = (acc[0] + acc[1]) + (acc[2] + acc[3])
                sumsqv = (acc2[0] + acc2[1]) + (acc2[2] + acc2[3])
                meanvec = sumv * inv_e
                varvec = sumsqv * inv_e - meanvec * meanvec
                rstdvec = _rsqrt(varvec + jnp.float32(1e-6))
                for i in range(LANES):
                    t = lrow + i
                    msp = jnp.full((LANES,), meanvec[i], jnp.float32)
                    rsp = jnp.full((LANES,), rstdvec[i], jnp.float32)
                    for k in range(nsub):
                        sl = pl.ds(t * emb + LANES * k, LANES)
                        xk = obuf[sl]
                        obuf[sl] = (xk - msp) * rsp * gs[k] + bs[k]

            plsc.parallel_loop(0, groups_per_half, 1)(group_body)

        def wave_step(w, p):
            for q in range(NSTREAM):
                gather_q(p, q).wait()

            @pl.when(w < n_waves - 1)
            def _():
                for q in range(NSTREAM):
                    gather_q(1 - p, q).start()

            @pl.when(w >= 1)
            def _():
                store_h(0, w - 1).wait()

            compute_half(p, 0)
            store_h(0, w).start()

            @pl.when(w >= 1)
            def _():
                store_h(1, w - 1).wait()

            compute_half(p, 1)
            store_h(1, w).start()

            @pl.when(w < n_waves - 2)
            def _():
                stage(w + 2, p)

        stage(0, 0)
        for q in range(NSTREAM):
            gather_q(0, q).start()
        stage(1, 1)

        def pair_body(i, _):
            wave_step(2 * i, 0)
            wave_step(2 * i + 1, 1)
            return 0

        lax.fori_loop(0, n_waves // 2, pair_body, 0)
        store_h(0, n_waves - 1).wait()
        store_h(1, n_waves - 1).wait()

    return sc_kernel


def kernel(src, seg, word_table, pos_table, gamma, beta):
    b, l = src.shape
    vocab, emb = word_table.shape
    n_pos = pos_table.shape[0]
    n = b * l
    flat_src = src.reshape(n).astype(jnp.int32)
    flat_seg = seg.reshape(n).astype(jnp.int32)
    sc = _make_sc_kernel(n, vocab, emb, n_pos)
    out = sc(flat_src, flat_seg, word_table, pos_table, gamma, beta)
    return out.reshape(b, l, emb)
